# baseline (speedup 1.0000x reference)
import functools

import jax
import jax.numpy as jnp
from jax import lax
from jax.experimental import pallas as pl
from jax.experimental.pallas import tpu as pltpu
from jax.experimental.pallas import tpu_sc as plsc

B, N, VP, VG, F = 4, 2048, 2562, 10000, 20000
VG_PAD = 10240
VP_PAD = 2688
F_PAD = 20224
FT = F_PAD // 16
RS = VG_PAD // 16
NB = 128
CB = 128
PT = (B * N) // 32
BIG = 1e18

@functools.cache
def _sc_mesh():
    return plsc.VectorSubcoreMesh(
        core_axis_name="c", subcore_axis_name="s",
        num_cores=2, num_subcores=16)


def _argmin_body(vpad, p_ref, g_ref, g2_ref, out_ref):
    pmat = p_ref[...]
    lane = lax.broadcasted_iota(jnp.int32, (NB, CB), 1)

    def step(j, carry):
        best_d, best_i = carry
        gmat = g_ref[0, :, pl.ds(j * CB, CB)]
        g2 = g2_ref[0, :, pl.ds(j * CB, CB)]
        prod = lax.dot_general(pmat, gmat, (((1,), (0,)), ((), ())),
                               precision=lax.Precision.HIGHEST,
                               preferred_element_type=jnp.float32)
        d = g2 - (prod + prod)
        idx = lane + j * CB
        m = d < best_d
        return jnp.where(m, d, best_d), jnp.where(m, idx, best_i)

    best_d = jnp.full((NB, CB), jnp.float32(3e38))
    best_i = jnp.zeros((NB, CB), jnp.int32)
    best_d, best_i = lax.fori_loop(0, vpad // CB, step, (best_d, best_i))
    mn = jnp.min(best_d, axis=1, keepdims=True)
    cand = jnp.where(best_d == mn, best_i, jnp.int32(0x7FFFFFFF))
    out_ref[...] = jnp.min(cand, axis=1, keepdims=True)


def _nn_argmin(p8, gstack, g2, vpad):
    nblk = N // NB
    grid = (B * nblk,)
    return pl.pallas_call(
        functools.partial(_argmin_body, vpad),
        grid=grid,
        in_specs=[
            pl.BlockSpec((NB, 8), lambda g: (g, 0)),
            pl.BlockSpec((1, 8, vpad), lambda g: (g // nblk, 0, 0)),
            pl.BlockSpec((1, 1, vpad), lambda g: (g // nblk, 0, 0)),
        ],
        out_specs=pl.BlockSpec((NB, 1), lambda g: (g, 0)),
        out_shape=jax.ShapeDtypeStruct((B * N, 1), jnp.int32),
    )(p8, gstack, g2)


def _vn_body(vx_h, vy_h, vz_h, fa_h, fb_h, fc_h, vn_out,
             tvx, tvy, tvz, tax, tay, taz, tfa, tfb, tfc, red, obuf, spacc):
    c = lax.axis_index("c")
    s = lax.axis_index("s")
    zero16 = jnp.zeros((16,), jnp.float32)

    for bl in range(2):
        b = 2 * c + bl
        if bl:
            plsc.subcore_barrier()
        pltpu.sync_copy(vx_h.at[pl.ds(b * VG_PAD, VG_PAD)], tvx)
        pltpu.sync_copy(vy_h.at[pl.ds(b * VG_PAD, VG_PAD)], tvy)
        pltpu.sync_copy(vz_h.at[pl.ds(b * VG_PAD, VG_PAD)], tvz)
        pltpu.sync_copy(fa_h.at[pl.ds(b * F_PAD + s * FT, FT)], tfa)
        pltpu.sync_copy(fb_h.at[pl.ds(b * F_PAD + s * FT, FT)], tfb)
        pltpu.sync_copy(fc_h.at[pl.ds(b * F_PAD + s * FT, FT)], tfc)

        def zstep(k, _):
            tax[pl.ds(k * 16, 16)] = zero16
            tay[pl.ds(k * 16, 16)] = zero16
            taz[pl.ds(k * 16, 16)] = zero16
            return 0
        lax.fori_loop(0, VG_PAD // 16, zstep, 0)

        def fstep(k, _):
            ia = tfa[pl.ds(k * 16, 16)]
            ib = tfb[pl.ds(k * 16, 16)]
            ic = tfc[pl.ds(k * 16, 16)]
            x0 = plsc.load_gather(tvx, [ia])
            y0 = plsc.load_gather(tvy, [ia])
            z0 = plsc.load_gather(tvz, [ia])
            x1 = plsc.load_gather(tvx, [ib])
            y1 = plsc.load_gather(tvy, [ib])
            z1 = plsc.load_gather(tvz, [ib])
            x2 = plsc.load_gather(tvx, [ic])
            y2 = plsc.load_gather(tvy, [ic])
            z2 = plsc.load_gather(tvz, [ic])
            ax_, ay_, az_ = x2 - x1, y2 - y1, z2 - z1
            bx_, by_, bz_ = x0 - x1, y0 - y1, z0 - z1
            nx = ay_ * bz_ - az_ * by_
            ny = az_ * bx_ - ax_ * bz_
            nz = ax_ * by_ - ay_ * bx_
            for ii in (ia, ib, ic):
                plsc.addupdate_scatter(tax, [ii], nx)
                plsc.addupdate_scatter(tay, [ii], ny)
                plsc.addupdate_scatter(taz, [ii], nz)
            return 0
        lax.fori_loop(0, FT // 16, fstep, 0)

        pltpu.sync_copy(tax, spacc.at[0, s])
        pltpu.sync_copy(tay, spacc.at[1, s])
        pltpu.sync_copy(taz, spacc.at[2, s])

        plsc.subcore_barrier()

        for comp in range(3):
            pltpu.sync_copy(spacc.at[comp, :, pl.ds(s * RS, RS)], red)

            def rstep(t, _):
                v = red[0, pl.ds(t * 16, 16)]
                for r in range(1, 16):
                    v = v + red[r, pl.ds(t * 16, 16)]
                obuf[pl.ds(t * 16, 16)] = v
                return 0
            lax.fori_loop(0, RS // 16, rstep, 0)
            pltpu.sync_copy(
                obuf,
                vn_out.at[pl.ds((b * 3 + comp) * VG_PAD + s * RS, RS)])


def _vertex_normals(vx, vy, vz, fa, fb, fc):
    return pl.kernel(
        _vn_body,
        out_type=jax.ShapeDtypeStruct((B * 3 * VG_PAD,), jnp.float32),
        mesh=_sc_mesh(),
        compiler_params=pltpu.CompilerParams(needs_layout_passes=False),
        scratch_types=[
            pltpu.VMEM((VG_PAD,), jnp.float32),
            pltpu.VMEM((VG_PAD,), jnp.float32),
            pltpu.VMEM((VG_PAD,), jnp.float32),
            pltpu.VMEM((VG_PAD,), jnp.float32),
            pltpu.VMEM((VG_PAD,), jnp.float32),
            pltpu.VMEM((VG_PAD,), jnp.float32),
            pltpu.VMEM((FT,), jnp.int32),
            pltpu.VMEM((FT,), jnp.int32),
            pltpu.VMEM((FT,), jnp.int32),
            pltpu.VMEM((16, RS), jnp.float32),
            pltpu.VMEM((RS,), jnp.float32),
            pltpu.VMEM_SHARED((3, 16, VG_PAD), jnp.float32),
        ],
    )(vx, vy, vz, fa, fb, fc)


def _rsqrt_nt(x):
    i = lax.bitcast_convert_type(x, jnp.int32)
    y = lax.bitcast_convert_type(jnp.int32(0x5F3759DF) - (i >> 1), jnp.float32)
    for _ in range(4):
        y = y * (jnp.float32(1.5) - jnp.float32(0.5) * x * y * y)
    return y


def _loss_body(vn_h, pvx_h, pvy_h, pvz_h, ppx_h, ppy_h, ppz_h, ig_h, ip_h,
               out_h, tnx, tny, tnz, tpx, tpy, tpz, idxg, idxp,
               px, py, pz, accbuf):
    c = lax.axis_index("c")
    s = lax.axis_index("s")
    w = c * 16 + s
    b = w // 8
    off = (w % 8) * PT

    pltpu.sync_copy(vn_h.at[pl.ds((b * 3 + 0) * VG_PAD, VG_PAD)], tnx)
    pltpu.sync_copy(vn_h.at[pl.ds((b * 3 + 1) * VG_PAD, VG_PAD)], tny)
    pltpu.sync_copy(vn_h.at[pl.ds((b * 3 + 2) * VG_PAD, VG_PAD)], tnz)
    pltpu.sync_copy(pvx_h.at[pl.ds(b * VP_PAD, VP_PAD)], tpx)
    pltpu.sync_copy(pvy_h.at[pl.ds(b * VP_PAD, VP_PAD)], tpy)
    pltpu.sync_copy(pvz_h.at[pl.ds(b * VP_PAD, VP_PAD)], tpz)
    pltpu.sync_copy(ig_h.at[pl.ds(b * N + off, PT)], idxg)
    pltpu.sync_copy(ip_h.at[pl.ds(b * N + off, PT)], idxp)
    pltpu.sync_copy(ppx_h.at[pl.ds(b * N + off, PT)], px)
    pltpu.sync_copy(ppy_h.at[pl.ds(b * N + off, PT)], py)
    pltpu.sync_copy(ppz_h.at[pl.ds(b * N + off, PT)], pz)

    def step(k, acc):
        g = idxg[pl.ds(k * 16, 16)]
        p = idxp[pl.ds(k * 16, 16)]
        nx = plsc.load_gather(tnx, [g])
        ny = plsc.load_gather(tny, [g])
        nz = plsc.load_gather(tnz, [g])
        vx = plsc.load_gather(tpx, [p])
        vy = plsc.load_gather(tpy, [p])
        vz = plsc.load_gather(tpz, [p])
        ex = px[pl.ds(k * 16, 16)] - vx
        ey = py[pl.ds(k * 16, 16)] - vy
        ez = pz[pl.ds(k * 16, 16)] - vz
        dot = ex * nx + ey * ny + ez * nz
        e2 = ex * ex + ey * ey + ez * ez
        n2 = nx * nx + ny * ny + nz * nz
        r = (jnp.abs(dot)
             * _rsqrt_nt(jnp.maximum(e2, jnp.float32(1e-24)))
             * _rsqrt_nt(jnp.maximum(n2, jnp.float32(1e-12))))
        return acc + r

    acc = lax.fori_loop(0, PT // 16, step, jnp.zeros((16,), jnp.float32))
    accbuf[...] = acc
    pltpu.sync_copy(accbuf, out_h.at[pl.ds(w * 16, 16)])


def _gather_loss(vn, pvx, pvy, pvz, ppx, ppy, ppz, ig, ip):
    return pl.kernel(
        _loss_body,
        out_type=jax.ShapeDtypeStruct((512,), jnp.float32),
        mesh=_sc_mesh(),
        compiler_params=pltpu.CompilerParams(needs_layout_passes=False),
        scratch_types=[
            pltpu.VMEM((VG_PAD,), jnp.float32),
            pltpu.VMEM((VG_PAD,), jnp.float32),
            pltpu.VMEM((VG_PAD,), jnp.float32),
            pltpu.VMEM((VP_PAD,), jnp.float32),
            pltpu.VMEM((VP_PAD,), jnp.float32),
            pltpu.VMEM((VP_PAD,), jnp.float32),
            pltpu.VMEM((PT,), jnp.int32),
            pltpu.VMEM((PT,), jnp.int32),
            pltpu.VMEM((PT,), jnp.float32),
            pltpu.VMEM((PT,), jnp.float32),
            pltpu.VMEM((PT,), jnp.float32),
            pltpu.VMEM((16,), jnp.float32),
        ],
    )(vn, pvx, pvy, pvz, ppx, ppy, ppz, ig, ip)


def kernel(pred_points, pred_vertices, gt_vertices, gt_faces):
    p8 = jnp.pad(pred_points.reshape(B * N, 3), ((0, 0), (0, 5)))

    def cand_stack(v, vpad):
        p = jnp.pad(v, ((0, 0), (0, vpad - v.shape[1]), (0, 0)),
                    constant_values=BIG)
        g = jnp.moveaxis(p, 2, 1)
        gstack = jnp.pad(g, ((0, 0), (0, 5), (0, 0)))
        g2 = jnp.sum(g * g, axis=1, keepdims=True)
        return gstack, g2

    gs, g2s = cand_stack(gt_vertices, VG_PAD)
    qs, q2s = cand_stack(pred_vertices, VP_PAD)

    ig = _nn_argmin(p8, gs, g2s, VG_PAD)
    ip = _nn_argmin(p8, qs, q2s, VP_PAD)

    vpad = jnp.pad(gt_vertices, ((0, 0), (0, VG_PAD - VG), (0, 0)))
    fpl = jnp.pad(gt_faces, ((0, 0), (0, F_PAD - F), (0, 0)),
                  constant_values=VG)
    vn = _vertex_normals(
        vpad[..., 0].reshape(-1), vpad[..., 1].reshape(-1),
        vpad[..., 2].reshape(-1),
        fpl[..., 0].reshape(-1), fpl[..., 1].reshape(-1),
        fpl[..., 2].reshape(-1))

    pvp = jnp.pad(pred_vertices, ((0, 0), (0, VP_PAD - VP), (0, 0)))

    partials = _gather_loss(
        vn, pvp[..., 0].reshape(-1), pvp[..., 1].reshape(-1),
        pvp[..., 2].reshape(-1),
        pred_points[..., 0].reshape(-1), pred_points[..., 1].reshape(-1),
        pred_points[..., 2].reshape(-1),
        ig.reshape(-1), ip.reshape(-1))
    return jnp.sum(partials) / jnp.float32(B * N)


# speedup vs baseline: 3.4342x; 3.4342x over previous
import functools

import jax
import jax.numpy as jnp
from jax import lax
from jax.experimental import pallas as pl
from jax.experimental.pallas import tpu as pltpu
from jax.experimental.pallas import tpu_sc as plsc

B, N, VP, VG, F = 4, 2048, 2562, 10000, 20000
VG_PAD = 10240
VP_PAD = 2688
F_PAD = 20224
FT = F_PAD // 16
RS = VG_PAD // 16
NB = 64
CB = 128
PT = (B * N) // 32
BIG = 1e18

@functools.cache
def _sc_mesh():
    return plsc.VectorSubcoreMesh(
        core_axis_name="c", subcore_axis_name="s",
        num_cores=2, num_subcores=16)


def _argmin_body(vpad, px_ref, py_ref, pz_ref,
                 gx_ref, gy_ref, gz_ref, out_ref):
    pxb = jnp.broadcast_to(px_ref[...], (NB, CB))
    pyb = jnp.broadcast_to(py_ref[...], (NB, CB))
    pzb = jnp.broadcast_to(pz_ref[...], (NB, CB))

    def step(j, carry):
        best_d, best_j = carry
        gx = gx_ref[0, :, pl.ds(j * CB, CB)]
        gy = gy_ref[0, :, pl.ds(j * CB, CB)]
        gz = gz_ref[0, :, pl.ds(j * CB, CB)]
        dx = pxb - gx
        dy = pyb - gy
        dz = pzb - gz
        d = (dx * dx + dy * dy) + dz * dz
        m = d < best_d
        return jnp.where(m, d, best_d), jnp.where(m, j, best_j)

    best_d = jnp.full((NB, CB), jnp.float32(3e38))
    best_j = jnp.zeros((NB, CB), jnp.int32)
    best_d, best_j = lax.fori_loop(0, vpad // CB, step, (best_d, best_j))
    lane = lax.broadcasted_iota(jnp.int32, (NB, CB), 1)
    best_i = best_j * CB + lane
    mn = jnp.min(best_d, axis=1, keepdims=True)
    cand = jnp.where(best_d == mn, best_i, jnp.int32(0x7FFFFFFF))
    out_ref[...] = jnp.min(cand, axis=1, keepdims=True)


def _nn_argmin(px, py, pz, gx, gy, gz, vpad):
    nblk = N // NB
    grid = (B * nblk,)
    p_spec = pl.BlockSpec((NB, 1), lambda g: (g, 0))
    g_spec = pl.BlockSpec((1, 1, vpad), lambda g: (g // nblk, 0, 0))
    return pl.pallas_call(
        functools.partial(_argmin_body, vpad),
        grid=grid,
        in_specs=[p_spec, p_spec, p_spec, g_spec, g_spec, g_spec],
        out_specs=pl.BlockSpec((NB, 1), lambda g: (g, 0)),
        out_shape=jax.ShapeDtypeStruct((B * N, 1), jnp.int32),
    )(px, py, pz, gx, gy, gz)


def _vn_body(vx_h, vy_h, vz_h, fa_h, fb_h, fc_h, vn_out,
             tvx, tvy, tvz, tax, tay, taz, tfa, tfb, tfc, red, obuf, spacc):
    c = lax.axis_index("c")
    s = lax.axis_index("s")
    zero16 = jnp.zeros((16,), jnp.float32)

    for bl in range(2):
        b = 2 * c + bl
        if bl:
            plsc.subcore_barrier()
        pltpu.sync_copy(vx_h.at[pl.ds(b * VG_PAD, VG_PAD)], tvx)
        pltpu.sync_copy(vy_h.at[pl.ds(b * VG_PAD, VG_PAD)], tvy)
        pltpu.sync_copy(vz_h.at[pl.ds(b * VG_PAD, VG_PAD)], tvz)
        pltpu.sync_copy(fa_h.at[pl.ds(b * F_PAD + s * FT, FT)], tfa)
        pltpu.sync_copy(fb_h.at[pl.ds(b * F_PAD + s * FT, FT)], tfb)
        pltpu.sync_copy(fc_h.at[pl.ds(b * F_PAD + s * FT, FT)], tfc)

        def zstep(k, _):
            tax[pl.ds(k * 16, 16)] = zero16
            tay[pl.ds(k * 16, 16)] = zero16
            taz[pl.ds(k * 16, 16)] = zero16
            return 0
        lax.fori_loop(0, VG_PAD // 16, zstep, 0)

        def fstep(k, _):
            ia = tfa[pl.ds(k * 16, 16)]
            ib = tfb[pl.ds(k * 16, 16)]
            ic = tfc[pl.ds(k * 16, 16)]
            x0 = plsc.load_gather(tvx, [ia])
            y0 = plsc.load_gather(tvy, [ia])
            z0 = plsc.load_gather(tvz, [ia])
            x1 = plsc.load_gather(tvx, [ib])
            y1 = plsc.load_gather(tvy, [ib])
            z1 = plsc.load_gather(tvz, [ib])
            x2 = plsc.load_gather(tvx, [ic])
            y2 = plsc.load_gather(tvy, [ic])
            z2 = plsc.load_gather(tvz, [ic])
            ax_, ay_, az_ = x2 - x1, y2 - y1, z2 - z1
            bx_, by_, bz_ = x0 - x1, y0 - y1, z0 - z1
            nx = ay_ * bz_ - az_ * by_
            ny = az_ * bx_ - ax_ * bz_
            nz = ax_ * by_ - ay_ * bx_
            for ii in (ia, ib, ic):
                plsc.addupdate_scatter(tax, [ii], nx)
                plsc.addupdate_scatter(tay, [ii], ny)
                plsc.addupdate_scatter(taz, [ii], nz)
            return 0
        lax.fori_loop(0, FT // 16, fstep, 0)

        pltpu.sync_copy(tax, spacc.at[0, s])
        pltpu.sync_copy(tay, spacc.at[1, s])
        pltpu.sync_copy(taz, spacc.at[2, s])

        plsc.subcore_barrier()

        for comp in range(3):
            pltpu.sync_copy(spacc.at[comp, :, pl.ds(s * RS, RS)], red)

            def rstep(t, _):
                v = red[0, pl.ds(t * 16, 16)]
                for r in range(1, 16):
                    v = v + red[r, pl.ds(t * 16, 16)]
                obuf[pl.ds(t * 16, 16)] = v
                return 0
            lax.fori_loop(0, RS // 16, rstep, 0)
            pltpu.sync_copy(
                obuf,
                vn_out.at[pl.ds((b * 3 + comp) * VG_PAD + s * RS, RS)])


def _vertex_normals(vx, vy, vz, fa, fb, fc):
    return pl.kernel(
        _vn_body,
        out_type=jax.ShapeDtypeStruct((B * 3 * VG_PAD,), jnp.float32),
        mesh=_sc_mesh(),
        compiler_params=pltpu.CompilerParams(needs_layout_passes=False),
        scratch_types=[
            pltpu.VMEM((VG_PAD,), jnp.float32),
            pltpu.VMEM((VG_PAD,), jnp.float32),
            pltpu.VMEM((VG_PAD,), jnp.float32),
            pltpu.VMEM((VG_PAD,), jnp.float32),
            pltpu.VMEM((VG_PAD,), jnp.float32),
            pltpu.VMEM((VG_PAD,), jnp.float32),
            pltpu.VMEM((FT,), jnp.int32),
            pltpu.VMEM((FT,), jnp.int32),
            pltpu.VMEM((FT,), jnp.int32),
            pltpu.VMEM((16, RS), jnp.float32),
            pltpu.VMEM((RS,), jnp.float32),
            pltpu.VMEM_SHARED((3, 16, VG_PAD), jnp.float32),
        ],
    )(vx, vy, vz, fa, fb, fc)


def _rsqrt_nt(x):
    i = lax.bitcast_convert_type(x, jnp.int32)
    y = lax.bitcast_convert_type(jnp.int32(0x5F3759DF) - (i >> 1), jnp.float32)
    for _ in range(4):
        y = y * (jnp.float32(1.5) - jnp.float32(0.5) * x * y * y)
    return y


def _loss_body(vn_h, pvx_h, pvy_h, pvz_h, ppx_h, ppy_h, ppz_h, ig_h, ip_h,
               out_h, tnx, tny, tnz, tpx, tpy, tpz, idxg, idxp,
               px, py, pz, accbuf):
    c = lax.axis_index("c")
    s = lax.axis_index("s")
    w = c * 16 + s
    b = w // 8
    off = (w % 8) * PT

    pltpu.sync_copy(vn_h.at[pl.ds((b * 3 + 0) * VG_PAD, VG_PAD)], tnx)
    pltpu.sync_copy(vn_h.at[pl.ds((b * 3 + 1) * VG_PAD, VG_PAD)], tny)
    pltpu.sync_copy(vn_h.at[pl.ds((b * 3 + 2) * VG_PAD, VG_PAD)], tnz)
    pltpu.sync_copy(pvx_h.at[pl.ds(b * VP_PAD, VP_PAD)], tpx)
    pltpu.sync_copy(pvy_h.at[pl.ds(b * VP_PAD, VP_PAD)], tpy)
    pltpu.sync_copy(pvz_h.at[pl.ds(b * VP_PAD, VP_PAD)], tpz)
    pltpu.sync_copy(ig_h.at[pl.ds(b * N + off, PT)], idxg)
    pltpu.sync_copy(ip_h.at[pl.ds(b * N + off, PT)], idxp)
    pltpu.sync_copy(ppx_h.at[pl.ds(b * N + off, PT)], px)
    pltpu.sync_copy(ppy_h.at[pl.ds(b * N + off, PT)], py)
    pltpu.sync_copy(ppz_h.at[pl.ds(b * N + off, PT)], pz)

    def step(k, acc):
        g = idxg[pl.ds(k * 16, 16)]
        p = idxp[pl.ds(k * 16, 16)]
        nx = plsc.load_gather(tnx, [g])
        ny = plsc.load_gather(tny, [g])
        nz = plsc.load_gather(tnz, [g])
        vx = plsc.load_gather(tpx, [p])
        vy = plsc.load_gather(tpy, [p])
        vz = plsc.load_gather(tpz, [p])
        ex = px[pl.ds(k * 16, 16)] - vx
        ey = py[pl.ds(k * 16, 16)] - vy
        ez = pz[pl.ds(k * 16, 16)] - vz
        dot = ex * nx + ey * ny + ez * nz
        e2 = ex * ex + ey * ey + ez * ez
        n2 = nx * nx + ny * ny + nz * nz
        r = (jnp.abs(dot)
             * _rsqrt_nt(jnp.maximum(e2, jnp.float32(1e-24)))
             * _rsqrt_nt(jnp.maximum(n2, jnp.float32(1e-12))))
        return acc + r

    acc = lax.fori_loop(0, PT // 16, step, jnp.zeros((16,), jnp.float32))
    accbuf[...] = acc
    pltpu.sync_copy(accbuf, out_h.at[pl.ds(w * 16, 16)])


def _gather_loss(vn, pvx, pvy, pvz, ppx, ppy, ppz, ig, ip):
    return pl.kernel(
        _loss_body,
        out_type=jax.ShapeDtypeStruct((512,), jnp.float32),
        mesh=_sc_mesh(),
        compiler_params=pltpu.CompilerParams(needs_layout_passes=False),
        scratch_types=[
            pltpu.VMEM((VG_PAD,), jnp.float32),
            pltpu.VMEM((VG_PAD,), jnp.float32),
            pltpu.VMEM((VG_PAD,), jnp.float32),
            pltpu.VMEM((VP_PAD,), jnp.float32),
            pltpu.VMEM((VP_PAD,), jnp.float32),
            pltpu.VMEM((VP_PAD,), jnp.float32),
            pltpu.VMEM((PT,), jnp.int32),
            pltpu.VMEM((PT,), jnp.int32),
            pltpu.VMEM((PT,), jnp.float32),
            pltpu.VMEM((PT,), jnp.float32),
            pltpu.VMEM((PT,), jnp.float32),
            pltpu.VMEM((16,), jnp.float32),
        ],
    )(vn, pvx, pvy, pvz, ppx, ppy, ppz, ig, ip)


def kernel(pred_points, pred_vertices, gt_vertices, gt_faces):
    ppx = pred_points[..., 0].reshape(B * N, 1)
    ppy = pred_points[..., 1].reshape(B * N, 1)
    ppz = pred_points[..., 2].reshape(B * N, 1)

    def cand_planes(v, vpad):
        p = jnp.pad(v, ((0, 0), (0, vpad - v.shape[1]), (0, 0)),
                    constant_values=BIG)
        return (p[..., 0][:, None, :], p[..., 1][:, None, :],
                p[..., 2][:, None, :])

    gx, gy, gz = cand_planes(gt_vertices, VG_PAD)
    qx, qy, qz = cand_planes(pred_vertices, VP_PAD)

    ig = _nn_argmin(ppx, ppy, ppz, gx, gy, gz, VG_PAD)
    ip = _nn_argmin(ppx, ppy, ppz, qx, qy, qz, VP_PAD)

    vpad = jnp.pad(gt_vertices, ((0, 0), (0, VG_PAD - VG), (0, 0)))
    fpl = jnp.pad(gt_faces, ((0, 0), (0, F_PAD - F), (0, 0)),
                  constant_values=VG)
    vn = _vertex_normals(
        vpad[..., 0].reshape(-1), vpad[..., 1].reshape(-1),
        vpad[..., 2].reshape(-1),
        fpl[..., 0].reshape(-1), fpl[..., 1].reshape(-1),
        fpl[..., 2].reshape(-1))

    pvp = jnp.pad(pred_vertices, ((0, 0), (0, VP_PAD - VP), (0, 0)))

    partials = _gather_loss(
        vn, pvp[..., 0].reshape(-1), pvp[..., 1].reshape(-1),
        pvp[..., 2].reshape(-1),
        pred_points[..., 0].reshape(-1), pred_points[..., 1].reshape(-1),
        pred_points[..., 2].reshape(-1),
        ig.reshape(-1), ip.reshape(-1))
    return jnp.sum(partials) / jnp.float32(B * N)


# speedup vs baseline: 3.7833x; 1.1016x over previous
import functools

import jax
import jax.numpy as jnp
from jax import lax
from jax.experimental import pallas as pl
from jax.experimental.pallas import tpu as pltpu
from jax.experimental.pallas import tpu_sc as plsc

B, N, VP, VG, F = 4, 2048, 2562, 10000, 20000
VG_PAD = 10240
VP_PAD = 2816
F_PAD = 20224
FT = F_PAD // 16
RS = VG_PAD // 16
NB = 64
CB = 128
PT = (B * N) // 32
BIG = 1e18

@functools.cache
def _sc_mesh():
    return plsc.VectorSubcoreMesh(
        core_axis_name="c", subcore_axis_name="s",
        num_cores=2, num_subcores=16)


def _argmin_body(vpad, px_ref, py_ref, pz_ref,
                 gx_ref, gy_ref, gz_ref, out_ref):
    pxb = jnp.broadcast_to(px_ref[...], (NB, CB))
    pyb = jnp.broadcast_to(py_ref[...], (NB, CB))
    pzb = jnp.broadcast_to(pz_ref[...], (NB, CB))

    def chunk_d(j):
        gx = gx_ref[0, :, pl.ds(j * CB, CB)]
        gy = gy_ref[0, :, pl.ds(j * CB, CB)]
        gz = gz_ref[0, :, pl.ds(j * CB, CB)]
        dx = pxb - gx
        dy = pyb - gy
        dz = pzb - gz
        return (dx * dx + dy * dy) + dz * dz

    def step(g, carry):
        best_d, best_j = carry
        j0 = 2 * g
        d0 = chunk_d(j0)
        d1 = chunk_d(j0 + 1)
        m01 = d1 < d0
        dp = jnp.where(m01, d1, d0)
        jp = jnp.where(m01, j0 + 1, j0)
        m = dp < best_d
        return jnp.where(m, dp, best_d), jnp.where(m, jp, best_j)

    best_d = jnp.full((NB, CB), jnp.float32(3e38))
    best_j = jnp.zeros((NB, CB), jnp.int32)
    best_d, best_j = lax.fori_loop(0, vpad // (2 * CB), step,
                                   (best_d, best_j))
    lane = lax.broadcasted_iota(jnp.int32, (NB, CB), 1)
    best_i = best_j * CB + lane
    mn = jnp.min(best_d, axis=1, keepdims=True)
    cand = jnp.where(best_d == mn, best_i, jnp.int32(0x7FFFFFFF))
    out_ref[...] = jnp.min(cand, axis=1, keepdims=True)


def _nn_argmin(px, py, pz, gx, gy, gz, vpad):
    nblk = N // NB
    grid = (B * nblk,)
    p_spec = pl.BlockSpec((NB, 1), lambda g: (g, 0))
    g_spec = pl.BlockSpec((1, 1, vpad), lambda g: (g // nblk, 0, 0))
    return pl.pallas_call(
        functools.partial(_argmin_body, vpad),
        grid=grid,
        in_specs=[p_spec, p_spec, p_spec, g_spec, g_spec, g_spec],
        out_specs=pl.BlockSpec((NB, 1), lambda g: (g, 0)),
        out_shape=jax.ShapeDtypeStruct((B * N, 1), jnp.int32),
    )(px, py, pz, gx, gy, gz)


def _vn_body(vx_h, vy_h, vz_h, fa_h, fb_h, fc_h, vn_out,
             tvx, tvy, tvz, tax, tay, taz, tfa, tfb, tfc, red, obuf, spacc):
    c = lax.axis_index("c")
    s = lax.axis_index("s")
    zero16 = jnp.zeros((16,), jnp.float32)

    for bl in range(2):
        b = 2 * c + bl
        if bl:
            plsc.subcore_barrier()
        pltpu.sync_copy(vx_h.at[pl.ds(b * VG_PAD, VG_PAD)], tvx)
        pltpu.sync_copy(vy_h.at[pl.ds(b * VG_PAD, VG_PAD)], tvy)
        pltpu.sync_copy(vz_h.at[pl.ds(b * VG_PAD, VG_PAD)], tvz)
        pltpu.sync_copy(fa_h.at[pl.ds(b * F_PAD + s * FT, FT)], tfa)
        pltpu.sync_copy(fb_h.at[pl.ds(b * F_PAD + s * FT, FT)], tfb)
        pltpu.sync_copy(fc_h.at[pl.ds(b * F_PAD + s * FT, FT)], tfc)

        def zstep(k, _):
            tax[pl.ds(k * 16, 16)] = zero16
            tay[pl.ds(k * 16, 16)] = zero16
            taz[pl.ds(k * 16, 16)] = zero16
            return 0
        lax.fori_loop(0, VG_PAD // 16, zstep, 0)

        def fstep(k, _):
            ia = tfa[pl.ds(k * 16, 16)]
            ib = tfb[pl.ds(k * 16, 16)]
            ic = tfc[pl.ds(k * 16, 16)]
            x0 = plsc.load_gather(tvx, [ia])
            y0 = plsc.load_gather(tvy, [ia])
            z0 = plsc.load_gather(tvz, [ia])
            x1 = plsc.load_gather(tvx, [ib])
            y1 = plsc.load_gather(tvy, [ib])
            z1 = plsc.load_gather(tvz, [ib])
            x2 = plsc.load_gather(tvx, [ic])
            y2 = plsc.load_gather(tvy, [ic])
            z2 = plsc.load_gather(tvz, [ic])
            ax_, ay_, az_ = x2 - x1, y2 - y1, z2 - z1
            bx_, by_, bz_ = x0 - x1, y0 - y1, z0 - z1
            nx = ay_ * bz_ - az_ * by_
            ny = az_ * bx_ - ax_ * bz_
            nz = ax_ * by_ - ay_ * bx_
            for ii in (ia, ib, ic):
                plsc.addupdate_scatter(tax, [ii], nx)
                plsc.addupdate_scatter(tay, [ii], ny)
                plsc.addupdate_scatter(taz, [ii], nz)
            return 0
        lax.fori_loop(0, FT // 16, fstep, 0)

        pltpu.sync_copy(tax, spacc.at[0, s])
        pltpu.sync_copy(tay, spacc.at[1, s])
        pltpu.sync_copy(taz, spacc.at[2, s])

        plsc.subcore_barrier()

        for comp in range(3):
            pltpu.sync_copy(spacc.at[comp, :, pl.ds(s * RS, RS)], red)

            def rstep(t, _):
                v = red[0, pl.ds(t * 16, 16)]
                for r in range(1, 16):
                    v = v + red[r, pl.ds(t * 16, 16)]
                obuf[pl.ds(t * 16, 16)] = v
                return 0
            lax.fori_loop(0, RS // 16, rstep, 0)
            pltpu.sync_copy(
                obuf,
                vn_out.at[pl.ds((b * 3 + comp) * VG_PAD + s * RS, RS)])


def _vertex_normals(vx, vy, vz, fa, fb, fc):
    return pl.kernel(
        _vn_body,
        out_type=jax.ShapeDtypeStruct((B * 3 * VG_PAD,), jnp.float32),
        mesh=_sc_mesh(),
        compiler_params=pltpu.CompilerParams(needs_layout_passes=False),
        scratch_types=[
            pltpu.VMEM((VG_PAD,), jnp.float32),
            pltpu.VMEM((VG_PAD,), jnp.float32),
            pltpu.VMEM((VG_PAD,), jnp.float32),
            pltpu.VMEM((VG_PAD,), jnp.float32),
            pltpu.VMEM((VG_PAD,), jnp.float32),
            pltpu.VMEM((VG_PAD,), jnp.float32),
            pltpu.VMEM((FT,), jnp.int32),
            pltpu.VMEM((FT,), jnp.int32),
            pltpu.VMEM((FT,), jnp.int32),
            pltpu.VMEM((16, RS), jnp.float32),
            pltpu.VMEM((RS,), jnp.float32),
            pltpu.VMEM_SHARED((3, 16, VG_PAD), jnp.float32),
        ],
    )(vx, vy, vz, fa, fb, fc)


def _rsqrt_nt(x):
    i = lax.bitcast_convert_type(x, jnp.int32)
    y = lax.bitcast_convert_type(jnp.int32(0x5F3759DF) - (i >> 1), jnp.float32)
    for _ in range(4):
        y = y * (jnp.float32(1.5) - jnp.float32(0.5) * x * y * y)
    return y


def _loss_body(vn_h, pvx_h, pvy_h, pvz_h, ppx_h, ppy_h, ppz_h, ig_h, ip_h,
               out_h, tnx, tny, tnz, tpx, tpy, tpz, idxg, idxp,
               px, py, pz, accbuf):
    c = lax.axis_index("c")
    s = lax.axis_index("s")
    w = c * 16 + s
    b = w // 8
    off = (w % 8) * PT

    pltpu.sync_copy(vn_h.at[pl.ds((b * 3 + 0) * VG_PAD, VG_PAD)], tnx)
    pltpu.sync_copy(vn_h.at[pl.ds((b * 3 + 1) * VG_PAD, VG_PAD)], tny)
    pltpu.sync_copy(vn_h.at[pl.ds((b * 3 + 2) * VG_PAD, VG_PAD)], tnz)
    pltpu.sync_copy(pvx_h.at[pl.ds(b * VP_PAD, VP_PAD)], tpx)
    pltpu.sync_copy(pvy_h.at[pl.ds(b * VP_PAD, VP_PAD)], tpy)
    pltpu.sync_copy(pvz_h.at[pl.ds(b * VP_PAD, VP_PAD)], tpz)
    pltpu.sync_copy(ig_h.at[pl.ds(b * N + off, PT)], idxg)
    pltpu.sync_copy(ip_h.at[pl.ds(b * N + off, PT)], idxp)
    pltpu.sync_copy(ppx_h.at[pl.ds(b * N + off, PT)], px)
    pltpu.sync_copy(ppy_h.at[pl.ds(b * N + off, PT)], py)
    pltpu.sync_copy(ppz_h.at[pl.ds(b * N + off, PT)], pz)

    def step(k, acc):
        g = idxg[pl.ds(k * 16, 16)]
        p = idxp[pl.ds(k * 16, 16)]
        nx = plsc.load_gather(tnx, [g])
        ny = plsc.load_gather(tny, [g])
        nz = plsc.load_gather(tnz, [g])
        vx = plsc.load_gather(tpx, [p])
        vy = plsc.load_gather(tpy, [p])
        vz = plsc.load_gather(tpz, [p])
        ex = px[pl.ds(k * 16, 16)] - vx
        ey = py[pl.ds(k * 16, 16)] - vy
        ez = pz[pl.ds(k * 16, 16)] - vz
        dot = ex * nx + ey * ny + ez * nz
        e2 = ex * ex + ey * ey + ez * ez
        n2 = nx * nx + ny * ny + nz * nz
        r = (jnp.abs(dot)
             * _rsqrt_nt(jnp.maximum(e2, jnp.float32(1e-24)))
             * _rsqrt_nt(jnp.maximum(n2, jnp.float32(1e-12))))
        return acc + r

    acc = lax.fori_loop(0, PT // 16, step, jnp.zeros((16,), jnp.float32))
    accbuf[...] = acc
    pltpu.sync_copy(accbuf, out_h.at[pl.ds(w * 16, 16)])


def _gather_loss(vn, pvx, pvy, pvz, ppx, ppy, ppz, ig, ip):
    return pl.kernel(
        _loss_body,
        out_type=jax.ShapeDtypeStruct((512,), jnp.float32),
        mesh=_sc_mesh(),
        compiler_params=pltpu.CompilerParams(needs_layout_passes=False),
        scratch_types=[
            pltpu.VMEM((VG_PAD,), jnp.float32),
            pltpu.VMEM((VG_PAD,), jnp.float32),
            pltpu.VMEM((VG_PAD,), jnp.float32),
            pltpu.VMEM((VP_PAD,), jnp.float32),
            pltpu.VMEM((VP_PAD,), jnp.float32),
            pltpu.VMEM((VP_PAD,), jnp.float32),
            pltpu.VMEM((PT,), jnp.int32),
            pltpu.VMEM((PT,), jnp.int32),
            pltpu.VMEM((PT,), jnp.float32),
            pltpu.VMEM((PT,), jnp.float32),
            pltpu.VMEM((PT,), jnp.float32),
            pltpu.VMEM((16,), jnp.float32),
        ],
    )(vn, pvx, pvy, pvz, ppx, ppy, ppz, ig, ip)


def kernel(pred_points, pred_vertices, gt_vertices, gt_faces):
    ppx = pred_points[..., 0].reshape(B * N, 1)
    ppy = pred_points[..., 1].reshape(B * N, 1)
    ppz = pred_points[..., 2].reshape(B * N, 1)

    def cand_planes(v, vpad):
        p = jnp.pad(v, ((0, 0), (0, vpad - v.shape[1]), (0, 0)),
                    constant_values=BIG)
        return (p[..., 0][:, None, :], p[..., 1][:, None, :],
                p[..., 2][:, None, :])

    gx, gy, gz = cand_planes(gt_vertices, VG_PAD)
    qx, qy, qz = cand_planes(pred_vertices, VP_PAD)

    ig = _nn_argmin(ppx, ppy, ppz, gx, gy, gz, VG_PAD)
    ip = _nn_argmin(ppx, ppy, ppz, qx, qy, qz, VP_PAD)

    vpad = jnp.pad(gt_vertices, ((0, 0), (0, VG_PAD - VG), (0, 0)))
    fpl = jnp.pad(gt_faces, ((0, 0), (0, F_PAD - F), (0, 0)),
                  constant_values=VG)
    vn = _vertex_normals(
        vpad[..., 0].reshape(-1), vpad[..., 1].reshape(-1),
        vpad[..., 2].reshape(-1),
        fpl[..., 0].reshape(-1), fpl[..., 1].reshape(-1),
        fpl[..., 2].reshape(-1))

    pvp = jnp.pad(pred_vertices, ((0, 0), (0, VP_PAD - VP), (0, 0)))

    partials = _gather_loss(
        vn, pvp[..., 0].reshape(-1), pvp[..., 1].reshape(-1),
        pvp[..., 2].reshape(-1),
        pred_points[..., 0].reshape(-1), pred_points[..., 1].reshape(-1),
        pred_points[..., 2].reshape(-1),
        ig.reshape(-1), ip.reshape(-1))
    return jnp.sum(partials) / jnp.float32(B * N)


# speedup vs baseline: 3.9733x; 1.0502x over previous
import functools

import jax
import jax.numpy as jnp
from jax import lax
from jax.experimental import pallas as pl
from jax.experimental.pallas import tpu as pltpu
from jax.experimental.pallas import tpu_sc as plsc

B, N, VP, VG, F = 4, 2048, 2562, 10000, 20000
VG_PAD = 10240
VP_PAD = 3072
F_PAD = 20224
FT = F_PAD // 16
RS = VG_PAD // 16
NB = 64
CB = 128
PT = (B * N) // 32
BIG = 1e18

@functools.cache
def _sc_mesh():
    return plsc.VectorSubcoreMesh(
        core_axis_name="c", subcore_axis_name="s",
        num_cores=2, num_subcores=16)


def _argmin_body(vpad, px_ref, py_ref, pz_ref,
                 gx_ref, gy_ref, gz_ref, out_ref):
    pxb = jnp.broadcast_to(px_ref[...], (NB, CB))
    pyb = jnp.broadcast_to(py_ref[...], (NB, CB))
    pzb = jnp.broadcast_to(pz_ref[...], (NB, CB))

    def chunk_d(j):
        gx = gx_ref[0, :, pl.ds(j * CB, CB)]
        gy = gy_ref[0, :, pl.ds(j * CB, CB)]
        gz = gz_ref[0, :, pl.ds(j * CB, CB)]
        dx = pxb - gx
        dy = pyb - gy
        dz = pzb - gz
        return (dx * dx + dy * dy) + dz * dz

    def step(g, carry):
        best_d, best_j = carry
        j0 = 4 * g
        d0 = chunk_d(j0)
        d1 = chunk_d(j0 + 1)
        d2 = chunk_d(j0 + 2)
        d3 = chunk_d(j0 + 3)
        m01 = d1 < d0
        da = jnp.where(m01, d1, d0)
        ja = jnp.where(m01, j0 + 1, j0)
        m23 = d3 < d2
        db = jnp.where(m23, d3, d2)
        jb = jnp.where(m23, j0 + 3, j0 + 2)
        mab = db < da
        dp = jnp.where(mab, db, da)
        jp = jnp.where(mab, jb, ja)
        m = dp < best_d
        return jnp.where(m, dp, best_d), jnp.where(m, jp, best_j)

    best_d = jnp.full((NB, CB), jnp.float32(3e38))
    best_j = jnp.zeros((NB, CB), jnp.int32)
    best_d, best_j = lax.fori_loop(0, vpad // (4 * CB), step,
                                   (best_d, best_j))
    lane = lax.broadcasted_iota(jnp.int32, (NB, CB), 1)
    best_i = best_j * CB + lane
    mn = jnp.min(best_d, axis=1, keepdims=True)
    cand = jnp.where(best_d == mn, best_i, jnp.int32(0x7FFFFFFF))
    out_ref[...] = jnp.min(cand, axis=1, keepdims=True)


def _nn_argmin(px, py, pz, gx, gy, gz, vpad):
    nblk = N // NB
    grid = (B * nblk,)
    p_spec = pl.BlockSpec((NB, 1), lambda g: (g, 0))
    g_spec = pl.BlockSpec((1, 1, vpad), lambda g: (g // nblk, 0, 0))
    return pl.pallas_call(
        functools.partial(_argmin_body, vpad),
        grid=grid,
        in_specs=[p_spec, p_spec, p_spec, g_spec, g_spec, g_spec],
        out_specs=pl.BlockSpec((NB, 1), lambda g: (g, 0)),
        out_shape=jax.ShapeDtypeStruct((B * N, 1), jnp.int32),
    )(px, py, pz, gx, gy, gz)


def _vn_body(vx_h, vy_h, vz_h, fa_h, fb_h, fc_h, vn_out,
             tvx, tvy, tvz, tax, tay, taz, tfa, tfb, tfc, red, obuf, spacc):
    c = lax.axis_index("c")
    s = lax.axis_index("s")
    zero16 = jnp.zeros((16,), jnp.float32)

    for bl in range(2):
        b = 2 * c + bl
        if bl:
            plsc.subcore_barrier()
        pltpu.sync_copy(vx_h.at[pl.ds(b * VG_PAD, VG_PAD)], tvx)
        pltpu.sync_copy(vy_h.at[pl.ds(b * VG_PAD, VG_PAD)], tvy)
        pltpu.sync_copy(vz_h.at[pl.ds(b * VG_PAD, VG_PAD)], tvz)
        pltpu.sync_copy(fa_h.at[pl.ds(b * F_PAD + s * FT, FT)], tfa)
        pltpu.sync_copy(fb_h.at[pl.ds(b * F_PAD + s * FT, FT)], tfb)
        pltpu.sync_copy(fc_h.at[pl.ds(b * F_PAD + s * FT, FT)], tfc)

        def zstep(k, _):
            tax[pl.ds(k * 16, 16)] = zero16
            tay[pl.ds(k * 16, 16)] = zero16
            taz[pl.ds(k * 16, 16)] = zero16
            return 0
        lax.fori_loop(0, VG_PAD // 16, zstep, 0)

        def fstep(k, _):
            ia = tfa[pl.ds(k * 16, 16)]
            ib = tfb[pl.ds(k * 16, 16)]
            ic = tfc[pl.ds(k * 16, 16)]
            x0 = plsc.load_gather(tvx, [ia])
            y0 = plsc.load_gather(tvy, [ia])
            z0 = plsc.load_gather(tvz, [ia])
            x1 = plsc.load_gather(tvx, [ib])
            y1 = plsc.load_gather(tvy, [ib])
            z1 = plsc.load_gather(tvz, [ib])
            x2 = plsc.load_gather(tvx, [ic])
            y2 = plsc.load_gather(tvy, [ic])
            z2 = plsc.load_gather(tvz, [ic])
            ax_, ay_, az_ = x2 - x1, y2 - y1, z2 - z1
            bx_, by_, bz_ = x0 - x1, y0 - y1, z0 - z1
            nx = ay_ * bz_ - az_ * by_
            ny = az_ * bx_ - ax_ * bz_
            nz = ax_ * by_ - ay_ * bx_
            for ii in (ia, ib, ic):
                plsc.addupdate_scatter(tax, [ii], nx)
                plsc.addupdate_scatter(tay, [ii], ny)
                plsc.addupdate_scatter(taz, [ii], nz)
            return 0
        lax.fori_loop(0, FT // 16, fstep, 0)

        pltpu.sync_copy(tax, spacc.at[0, s])
        pltpu.sync_copy(tay, spacc.at[1, s])
        pltpu.sync_copy(taz, spacc.at[2, s])

        plsc.subcore_barrier()

        for comp in range(3):
            pltpu.sync_copy(spacc.at[comp, :, pl.ds(s * RS, RS)], red)

            def rstep(t, _):
                v = red[0, pl.ds(t * 16, 16)]
                for r in range(1, 16):
                    v = v + red[r, pl.ds(t * 16, 16)]
                obuf[pl.ds(t * 16, 16)] = v
                return 0
            lax.fori_loop(0, RS // 16, rstep, 0)
            pltpu.sync_copy(
                obuf,
                vn_out.at[pl.ds((b * 3 + comp) * VG_PAD + s * RS, RS)])


def _vertex_normals(vx, vy, vz, fa, fb, fc):
    return pl.kernel(
        _vn_body,
        out_type=jax.ShapeDtypeStruct((B * 3 * VG_PAD,), jnp.float32),
        mesh=_sc_mesh(),
        compiler_params=pltpu.CompilerParams(needs_layout_passes=False),
        scratch_types=[
            pltpu.VMEM((VG_PAD,), jnp.float32),
            pltpu.VMEM((VG_PAD,), jnp.float32),
            pltpu.VMEM((VG_PAD,), jnp.float32),
            pltpu.VMEM((VG_PAD,), jnp.float32),
            pltpu.VMEM((VG_PAD,), jnp.float32),
            pltpu.VMEM((VG_PAD,), jnp.float32),
            pltpu.VMEM((FT,), jnp.int32),
            pltpu.VMEM((FT,), jnp.int32),
            pltpu.VMEM((FT,), jnp.int32),
            pltpu.VMEM((16, RS), jnp.float32),
            pltpu.VMEM((RS,), jnp.float32),
            pltpu.VMEM_SHARED((3, 16, VG_PAD), jnp.float32),
        ],
    )(vx, vy, vz, fa, fb, fc)


def _rsqrt_nt(x):
    i = lax.bitcast_convert_type(x, jnp.int32)
    y = lax.bitcast_convert_type(jnp.int32(0x5F3759DF) - (i >> 1), jnp.float32)
    for _ in range(4):
        y = y * (jnp.float32(1.5) - jnp.float32(0.5) * x * y * y)
    return y


def _loss_body(vn_h, pvx_h, pvy_h, pvz_h, ppx_h, ppy_h, ppz_h, ig_h, ip_h,
               out_h, tnx, tny, tnz, tpx, tpy, tpz, idxg, idxp,
               px, py, pz, accbuf):
    c = lax.axis_index("c")
    s = lax.axis_index("s")
    w = c * 16 + s
    b = w // 8
    off = (w % 8) * PT

    pltpu.sync_copy(vn_h.at[pl.ds((b * 3 + 0) * VG_PAD, VG_PAD)], tnx)
    pltpu.sync_copy(vn_h.at[pl.ds((b * 3 + 1) * VG_PAD, VG_PAD)], tny)
    pltpu.sync_copy(vn_h.at[pl.ds((b * 3 + 2) * VG_PAD, VG_PAD)], tnz)
    pltpu.sync_copy(pvx_h.at[pl.ds(b * VP_PAD, VP_PAD)], tpx)
    pltpu.sync_copy(pvy_h.at[pl.ds(b * VP_PAD, VP_PAD)], tpy)
    pltpu.sync_copy(pvz_h.at[pl.ds(b * VP_PAD, VP_PAD)], tpz)
    pltpu.sync_copy(ig_h.at[pl.ds(b * N + off, PT)], idxg)
    pltpu.sync_copy(ip_h.at[pl.ds(b * N + off, PT)], idxp)
    pltpu.sync_copy(ppx_h.at[pl.ds(b * N + off, PT)], px)
    pltpu.sync_copy(ppy_h.at[pl.ds(b * N + off, PT)], py)
    pltpu.sync_copy(ppz_h.at[pl.ds(b * N + off, PT)], pz)

    def step(k, acc):
        g = idxg[pl.ds(k * 16, 16)]
        p = idxp[pl.ds(k * 16, 16)]
        nx = plsc.load_gather(tnx, [g])
        ny = plsc.load_gather(tny, [g])
        nz = plsc.load_gather(tnz, [g])
        vx = plsc.load_gather(tpx, [p])
        vy = plsc.load_gather(tpy, [p])
        vz = plsc.load_gather(tpz, [p])
        ex = px[pl.ds(k * 16, 16)] - vx
        ey = py[pl.ds(k * 16, 16)] - vy
        ez = pz[pl.ds(k * 16, 16)] - vz
        dot = ex * nx + ey * ny + ez * nz
        e2 = ex * ex + ey * ey + ez * ez
        n2 = nx * nx + ny * ny + nz * nz
        r = (jnp.abs(dot)
             * _rsqrt_nt(jnp.maximum(e2, jnp.float32(1e-24)))
             * _rsqrt_nt(jnp.maximum(n2, jnp.float32(1e-12))))
        return acc + r

    acc = lax.fori_loop(0, PT // 16, step, jnp.zeros((16,), jnp.float32))
    accbuf[...] = acc
    pltpu.sync_copy(accbuf, out_h.at[pl.ds(w * 16, 16)])


def _gather_loss(vn, pvx, pvy, pvz, ppx, ppy, ppz, ig, ip):
    return pl.kernel(
        _loss_body,
        out_type=jax.ShapeDtypeStruct((512,), jnp.float32),
        mesh=_sc_mesh(),
        compiler_params=pltpu.CompilerParams(needs_layout_passes=False),
        scratch_types=[
            pltpu.VMEM((VG_PAD,), jnp.float32),
            pltpu.VMEM((VG_PAD,), jnp.float32),
            pltpu.VMEM((VG_PAD,), jnp.float32),
            pltpu.VMEM((VP_PAD,), jnp.float32),
            pltpu.VMEM((VP_PAD,), jnp.float32),
            pltpu.VMEM((VP_PAD,), jnp.float32),
            pltpu.VMEM((PT,), jnp.int32),
            pltpu.VMEM((PT,), jnp.int32),
            pltpu.VMEM((PT,), jnp.float32),
            pltpu.VMEM((PT,), jnp.float32),
            pltpu.VMEM((PT,), jnp.float32),
            pltpu.VMEM((16,), jnp.float32),
        ],
    )(vn, pvx, pvy, pvz, ppx, ppy, ppz, ig, ip)


def kernel(pred_points, pred_vertices, gt_vertices, gt_faces):
    ppx = pred_points[..., 0].reshape(B * N, 1)
    ppy = pred_points[..., 1].reshape(B * N, 1)
    ppz = pred_points[..., 2].reshape(B * N, 1)

    def cand_planes(v, vpad):
        p = jnp.pad(v, ((0, 0), (0, vpad - v.shape[1]), (0, 0)),
                    constant_values=BIG)
        return (p[..., 0][:, None, :], p[..., 1][:, None, :],
                p[..., 2][:, None, :])

    vpad = jnp.pad(gt_vertices, ((0, 0), (0, VG_PAD - VG), (0, 0)))
    fpl = jnp.pad(gt_faces, ((0, 0), (0, F_PAD - F), (0, 0)),
                  constant_values=VG)
    vn = _vertex_normals(
        vpad[..., 0].reshape(-1), vpad[..., 1].reshape(-1),
        vpad[..., 2].reshape(-1),
        fpl[..., 0].reshape(-1), fpl[..., 1].reshape(-1),
        fpl[..., 2].reshape(-1))

    gx, gy, gz = cand_planes(gt_vertices, VG_PAD)
    qx, qy, qz = cand_planes(pred_vertices, VP_PAD)

    ig = _nn_argmin(ppx, ppy, ppz, gx, gy, gz, VG_PAD)
    ip = _nn_argmin(ppx, ppy, ppz, qx, qy, qz, VP_PAD)

    pvp = jnp.pad(pred_vertices, ((0, 0), (0, VP_PAD - VP), (0, 0)))

    partials = _gather_loss(
        vn, pvp[..., 0].reshape(-1), pvp[..., 1].reshape(-1),
        pvp[..., 2].reshape(-1),
        pred_points[..., 0].reshape(-1), pred_points[..., 1].reshape(-1),
        pred_points[..., 2].reshape(-1),
        ig.reshape(-1), ip.reshape(-1))
    return jnp.sum(partials) / jnp.float32(B * N)


# speedup vs baseline: 4.3538x; 1.0958x over previous
import functools

import jax
import jax.numpy as jnp
from jax import lax
from jax.experimental import pallas as pl
from jax.experimental.pallas import tpu as pltpu
from jax.experimental.pallas import tpu_sc as plsc

B, N, VP, VG, F = 4, 2048, 2562, 10000, 20000
VG_PAD = 10240
VP_PAD = 3072
F_PAD = 20224
FT = F_PAD // 16
RS = VG_PAD // 16
NB = 64
CB = 128
PT = (B * N) // 32
BIG = 1e18

@functools.cache
def _sc_mesh():
    return plsc.VectorSubcoreMesh(
        core_axis_name="c", subcore_axis_name="s",
        num_cores=2, num_subcores=16)


def _argmin_body(vpads, px_ref, py_ref, pz_ref,
                 gx_ref, gy_ref, gz_ref, ig_ref, ip_ref):
    pxb = jnp.broadcast_to(px_ref[...], (NB, CB))
    pyb = jnp.broadcast_to(py_ref[...], (NB, CB))
    pzb = jnp.broadcast_to(pz_ref[...], (NB, CB))

    def chunk_d(j):
        gx = gx_ref[0, :, pl.ds(j * CB, CB)]
        gy = gy_ref[0, :, pl.ds(j * CB, CB)]
        gz = gz_ref[0, :, pl.ds(j * CB, CB)]
        dx = pxb - gx
        dy = pyb - gy
        dz = pzb - gz
        return (dx * dx + dy * dy) + dz * dz

    def step(g, carry):
        best_d, best_j = carry
        j0 = 4 * g
        d0 = chunk_d(j0)
        d1 = chunk_d(j0 + 1)
        d2 = chunk_d(j0 + 2)
        d3 = chunk_d(j0 + 3)
        m01 = d1 < d0
        da = jnp.where(m01, d1, d0)
        ja = jnp.where(m01, j0 + 1, j0)
        m23 = d3 < d2
        db = jnp.where(m23, d3, d2)
        jb = jnp.where(m23, j0 + 3, j0 + 2)
        mab = db < da
        dp = jnp.where(mab, db, da)
        jp = jnp.where(mab, jb, ja)
        m = dp < best_d
        return jnp.where(m, dp, best_d), jnp.where(m, jp, best_j)

    lane = lax.broadcasted_iota(jnp.int32, (NB, CB), 1)
    base = 0
    for vpad, out_ref in zip(vpads, (ig_ref, ip_ref)):
        best_d = jnp.full((NB, CB), jnp.float32(3e38))
        best_j = jnp.zeros((NB, CB), jnp.int32)
        bj = base // CB
        best_d, best_j = lax.fori_loop(
            bj // 4, (base + vpad) // (4 * CB), step, (best_d, best_j))
        best_i = (best_j - bj) * CB + lane
        mn = jnp.min(best_d, axis=1, keepdims=True)
        cand = jnp.where(best_d == mn, best_i, jnp.int32(0x7FFFFFFF))
        out_ref[...] = jnp.min(cand, axis=1, keepdims=True)
        base += vpad


def _nn_argmin2(px, py, pz, gx, gy, gz):
    nblk = N // NB
    grid = (B * nblk,)
    vall = VG_PAD + VP_PAD
    p_spec = pl.BlockSpec((NB, 1), lambda g: (g, 0))
    g_spec = pl.BlockSpec((1, 1, vall), lambda g: (g // nblk, 0, 0))
    o_spec = pl.BlockSpec((NB, 1), lambda g: (g, 0))
    return pl.pallas_call(
        functools.partial(_argmin_body, (VG_PAD, VP_PAD)),
        grid=grid,
        in_specs=[p_spec, p_spec, p_spec, g_spec, g_spec, g_spec],
        out_specs=(o_spec, o_spec),
        out_shape=(jax.ShapeDtypeStruct((B * N, 1), jnp.int32),
                   jax.ShapeDtypeStruct((B * N, 1), jnp.int32)),
    )(px, py, pz, gx, gy, gz)


def _vn_body(vx_h, vy_h, vz_h, fa_h, fb_h, fc_h, vn_out,
             tvx, tvy, tvz, tax, tay, taz, tfa, tfb, tfc, red, obuf, spacc):
    c = lax.axis_index("c")
    s = lax.axis_index("s")
    zero16 = jnp.zeros((16,), jnp.float32)

    for bl in range(2):
        b = 2 * c + bl
        if bl:
            plsc.subcore_barrier()
        pltpu.sync_copy(vx_h.at[pl.ds(b * VG_PAD, VG_PAD)], tvx)
        pltpu.sync_copy(vy_h.at[pl.ds(b * VG_PAD, VG_PAD)], tvy)
        pltpu.sync_copy(vz_h.at[pl.ds(b * VG_PAD, VG_PAD)], tvz)
        pltpu.sync_copy(fa_h.at[pl.ds(b * F_PAD + s * FT, FT)], tfa)
        pltpu.sync_copy(fb_h.at[pl.ds(b * F_PAD + s * FT, FT)], tfb)
        pltpu.sync_copy(fc_h.at[pl.ds(b * F_PAD + s * FT, FT)], tfc)

        def zstep(k, _):
            tax[pl.ds(k * 16, 16)] = zero16
            tay[pl.ds(k * 16, 16)] = zero16
            taz[pl.ds(k * 16, 16)] = zero16
            return 0
        lax.fori_loop(0, VG_PAD // 16, zstep, 0)

        def fstep(k, _):
            ia = tfa[pl.ds(k * 16, 16)]
            ib = tfb[pl.ds(k * 16, 16)]
            ic = tfc[pl.ds(k * 16, 16)]
            x0 = plsc.load_gather(tvx, [ia])
            y0 = plsc.load_gather(tvy, [ia])
            z0 = plsc.load_gather(tvz, [ia])
            x1 = plsc.load_gather(tvx, [ib])
            y1 = plsc.load_gather(tvy, [ib])
            z1 = plsc.load_gather(tvz, [ib])
            x2 = plsc.load_gather(tvx, [ic])
            y2 = plsc.load_gather(tvy, [ic])
            z2 = plsc.load_gather(tvz, [ic])
            ax_, ay_, az_ = x2 - x1, y2 - y1, z2 - z1
            bx_, by_, bz_ = x0 - x1, y0 - y1, z0 - z1
            nx = ay_ * bz_ - az_ * by_
            ny = az_ * bx_ - ax_ * bz_
            nz = ax_ * by_ - ay_ * bx_
            for ii in (ia, ib, ic):
                plsc.addupdate_scatter(tax, [ii], nx)
                plsc.addupdate_scatter(tay, [ii], ny)
                plsc.addupdate_scatter(taz, [ii], nz)
            return 0
        lax.fori_loop(0, FT // 16, fstep, 0)

        pltpu.sync_copy(tax, spacc.at[0, s])
        pltpu.sync_copy(tay, spacc.at[1, s])
        pltpu.sync_copy(taz, spacc.at[2, s])

        plsc.subcore_barrier()

        for comp in range(3):
            pltpu.sync_copy(spacc.at[comp, :, pl.ds(s * RS, RS)], red)

            def rstep(t, _):
                v = red[0, pl.ds(t * 16, 16)]
                for r in range(1, 16):
                    v = v + red[r, pl.ds(t * 16, 16)]
                obuf[pl.ds(t * 16, 16)] = v
                return 0
            lax.fori_loop(0, RS // 16, rstep, 0)
            pltpu.sync_copy(
                obuf,
                vn_out.at[pl.ds((b * 3 + comp) * VG_PAD + s * RS, RS)])


def _vertex_normals(vx, vy, vz, fa, fb, fc):
    return pl.kernel(
        _vn_body,
        out_type=jax.ShapeDtypeStruct((B * 3 * VG_PAD,), jnp.float32),
        mesh=_sc_mesh(),
        compiler_params=pltpu.CompilerParams(needs_layout_passes=False),
        scratch_types=[
            pltpu.VMEM((VG_PAD,), jnp.float32),
            pltpu.VMEM((VG_PAD,), jnp.float32),
            pltpu.VMEM((VG_PAD,), jnp.float32),
            pltpu.VMEM((VG_PAD,), jnp.float32),
            pltpu.VMEM((VG_PAD,), jnp.float32),
            pltpu.VMEM((VG_PAD,), jnp.float32),
            pltpu.VMEM((FT,), jnp.int32),
            pltpu.VMEM((FT,), jnp.int32),
            pltpu.VMEM((FT,), jnp.int32),
            pltpu.VMEM((16, RS), jnp.float32),
            pltpu.VMEM((RS,), jnp.float32),
            pltpu.VMEM_SHARED((3, 16, VG_PAD), jnp.float32),
        ],
    )(vx, vy, vz, fa, fb, fc)


def _rsqrt_nt(x):
    i = lax.bitcast_convert_type(x, jnp.int32)
    y = lax.bitcast_convert_type(jnp.int32(0x5F3759DF) - (i >> 1), jnp.float32)
    for _ in range(4):
        y = y * (jnp.float32(1.5) - jnp.float32(0.5) * x * y * y)
    return y


def _loss_body(vn_h, pvx_h, pvy_h, pvz_h, ppx_h, ppy_h, ppz_h, ig_h, ip_h,
               out_h, tnx, tny, tnz, tpx, tpy, tpz, idxg, idxp,
               px, py, pz, accbuf):
    c = lax.axis_index("c")
    s = lax.axis_index("s")
    w = c * 16 + s
    b = w // 8
    off = (w % 8) * PT

    pltpu.sync_copy(vn_h.at[pl.ds((b * 3 + 0) * VG_PAD, VG_PAD)], tnx)
    pltpu.sync_copy(vn_h.at[pl.ds((b * 3 + 1) * VG_PAD, VG_PAD)], tny)
    pltpu.sync_copy(vn_h.at[pl.ds((b * 3 + 2) * VG_PAD, VG_PAD)], tnz)
    pltpu.sync_copy(pvx_h.at[pl.ds(b * VP_PAD, VP_PAD)], tpx)
    pltpu.sync_copy(pvy_h.at[pl.ds(b * VP_PAD, VP_PAD)], tpy)
    pltpu.sync_copy(pvz_h.at[pl.ds(b * VP_PAD, VP_PAD)], tpz)
    pltpu.sync_copy(ig_h.at[pl.ds(b * N + off, PT)], idxg)
    pltpu.sync_copy(ip_h.at[pl.ds(b * N + off, PT)], idxp)
    pltpu.sync_copy(ppx_h.at[pl.ds(b * N + off, PT)], px)
    pltpu.sync_copy(ppy_h.at[pl.ds(b * N + off, PT)], py)
    pltpu.sync_copy(ppz_h.at[pl.ds(b * N + off, PT)], pz)

    def step(k, acc):
        g = idxg[pl.ds(k * 16, 16)]
        p = idxp[pl.ds(k * 16, 16)]
        nx = plsc.load_gather(tnx, [g])
        ny = plsc.load_gather(tny, [g])
        nz = plsc.load_gather(tnz, [g])
        vx = plsc.load_gather(tpx, [p])
        vy = plsc.load_gather(tpy, [p])
        vz = plsc.load_gather(tpz, [p])
        ex = px[pl.ds(k * 16, 16)] - vx
        ey = py[pl.ds(k * 16, 16)] - vy
        ez = pz[pl.ds(k * 16, 16)] - vz
        dot = ex * nx + ey * ny + ez * nz
        e2 = ex * ex + ey * ey + ez * ez
        n2 = nx * nx + ny * ny + nz * nz
        r = (jnp.abs(dot)
             * _rsqrt_nt(jnp.maximum(e2, jnp.float32(1e-24)))
             * _rsqrt_nt(jnp.maximum(n2, jnp.float32(1e-12))))
        return acc + r

    acc = lax.fori_loop(0, PT // 16, step, jnp.zeros((16,), jnp.float32))
    accbuf[...] = acc
    pltpu.sync_copy(accbuf, out_h.at[pl.ds(w * 16, 16)])


def _gather_loss(vn, pvx, pvy, pvz, ppx, ppy, ppz, ig, ip):
    return pl.kernel(
        _loss_body,
        out_type=jax.ShapeDtypeStruct((512,), jnp.float32),
        mesh=_sc_mesh(),
        compiler_params=pltpu.CompilerParams(needs_layout_passes=False),
        scratch_types=[
            pltpu.VMEM((VG_PAD,), jnp.float32),
            pltpu.VMEM((VG_PAD,), jnp.float32),
            pltpu.VMEM((VG_PAD,), jnp.float32),
            pltpu.VMEM((VP_PAD,), jnp.float32),
            pltpu.VMEM((VP_PAD,), jnp.float32),
            pltpu.VMEM((VP_PAD,), jnp.float32),
            pltpu.VMEM((PT,), jnp.int32),
            pltpu.VMEM((PT,), jnp.int32),
            pltpu.VMEM((PT,), jnp.float32),
            pltpu.VMEM((PT,), jnp.float32),
            pltpu.VMEM((PT,), jnp.float32),
            pltpu.VMEM((16,), jnp.float32),
        ],
    )(vn, pvx, pvy, pvz, ppx, ppy, ppz, ig, ip)


def kernel(pred_points, pred_vertices, gt_vertices, gt_faces):
    ppx = pred_points[..., 0].reshape(B * N, 1)
    ppy = pred_points[..., 1].reshape(B * N, 1)
    ppz = pred_points[..., 2].reshape(B * N, 1)

    def cand_planes(v, vpad):
        p = jnp.pad(v, ((0, 0), (0, vpad - v.shape[1]), (0, 0)),
                    constant_values=BIG)
        return (p[..., 0][:, None, :], p[..., 1][:, None, :],
                p[..., 2][:, None, :])

    vpad = jnp.pad(gt_vertices, ((0, 0), (0, VG_PAD - VG), (0, 0)))
    fpl = jnp.pad(gt_faces, ((0, 0), (0, F_PAD - F), (0, 0)),
                  constant_values=VG)
    vn = _vertex_normals(
        vpad[..., 0].reshape(-1), vpad[..., 1].reshape(-1),
        vpad[..., 2].reshape(-1),
        fpl[..., 0].reshape(-1), fpl[..., 1].reshape(-1),
        fpl[..., 2].reshape(-1))

    gx, gy, gz = cand_planes(gt_vertices, VG_PAD)
    qx, qy, qz = cand_planes(pred_vertices, VP_PAD)

    ig, ip = _nn_argmin2(ppx, ppy, ppz,
                         jnp.concatenate([gx, qx], axis=2),
                         jnp.concatenate([gy, qy], axis=2),
                         jnp.concatenate([gz, qz], axis=2))

    pvp = jnp.pad(pred_vertices, ((0, 0), (0, VP_PAD - VP), (0, 0)))

    partials = _gather_loss(
        vn, pvp[..., 0].reshape(-1), pvp[..., 1].reshape(-1),
        pvp[..., 2].reshape(-1),
        pred_points[..., 0].reshape(-1), pred_points[..., 1].reshape(-1),
        pred_points[..., 2].reshape(-1),
        ig.reshape(-1), ip.reshape(-1))
    return jnp.sum(partials) / jnp.float32(B * N)
